# Initial kernel scaffold; baseline (speedup 1.0000x reference)
import jax
import jax.numpy as jnp
from jax.experimental import pallas as pl


def kernel(edge_index_dr, edge_weight_dr, edge_index_ds, edge_weight_ds, edge_index_drds, edges_ori, params):
    raise NotImplementedError("write your pallas kernel here")


# speedup vs baseline: 11.2326x; 11.2326x over previous
import functools

import jax
import jax.numpy as jnp
import numpy as np
from jax import lax
from jax.experimental import pallas as pl
from jax.experimental.pallas import tpu as pltpu

N_DR = 2048
N_DS = 2048
N_DD = 4096
H1 = 512
H2 = 128
NH = 8
NL = 2
DFUI = 384
DK = DFUI // NH
BATCH = 1024

_INTERPRET = False


def _gelu(x):
    return x * 0.5 * (1.0 + lax.erf(x * np.float32(1.0 / np.sqrt(2.0))))


def _bn(x, g, b):
    return x * np.float32(1.0 / np.sqrt(1.0 + 1e-5)) * g + b


def _ln(x, g, b):
    m = jnp.mean(x, axis=-1, keepdims=True)
    v = jnp.mean((x - m) ** 2, axis=-1, keepdims=True)
    return (x - m) * lax.rsqrt(v + 1e-5) * g + b


def _mm_body(x_ref, w_ref, o_ref):
    o_ref[...] = jnp.dot(x_ref[...], w_ref[...],
                         preferred_element_type=jnp.float32)


def _matmul(x, w, bm=256):
    m, k = x.shape
    _, n = w.shape
    return pl.pallas_call(
        _mm_body,
        grid=(m // bm,),
        in_specs=[
            pl.BlockSpec((bm, k), lambda i: (i, 0)),
            pl.BlockSpec((k, n), lambda i: (0, 0)),
        ],
        out_specs=pl.BlockSpec((bm, n), lambda i: (i, 0)),
        out_shape=jax.ShapeDtypeStruct((m, n), jnp.float32),
        interpret=_INTERPRET,
    )(x, w)


def _sage_body(a_ref, y_ref, yrow_ref, deg_ref, b_ref, o_ref, *, relu):
    acc = jnp.dot(a_ref[...], y_ref[...], preferred_element_type=jnp.float32)
    inv = 1.0 / (deg_ref[...] + 1.0)
    out = (acc + yrow_ref[...]) * inv + b_ref[...]
    if relu:
        out = jnp.maximum(out, 0.0)
    o_ref[...] = out


def _sage_combine(a, y, deg2d, bias2d, relu, bm=256):
    n, f = y.shape
    return pl.pallas_call(
        functools.partial(_sage_body, relu=relu),
        grid=(n // bm,),
        in_specs=[
            pl.BlockSpec((bm, n), lambda i: (i, 0)),
            pl.BlockSpec((n, f), lambda i: (0, 0)),
            pl.BlockSpec((bm, f), lambda i: (i, 0)),
            pl.BlockSpec((bm, 1), lambda i: (i, 0)),
            pl.BlockSpec((1, f), lambda i: (0, 0)),
        ],
        out_specs=pl.BlockSpec((bm, f), lambda i: (i, 0)),
        out_shape=jax.ShapeDtypeStruct((n, f), jnp.float32),
        interpret=_INTERPRET,
    )(a, y, y, deg2d, bias2d)


def _zzt_body(za_ref, zb_ref, o_ref, *, sigmoid):
    acc = lax.dot_general(za_ref[...], zb_ref[...],
                          (((1,), (1,)), ((), ())),
                          preferred_element_type=jnp.float32)
    if sigmoid:
        acc = 1.0 / (1.0 + jnp.exp(-acc))
    o_ref[...] = acc


def _zzt(za, zb, sigmoid=False, bm=256):
    ma, f = za.shape
    mb, _ = zb.shape
    return pl.pallas_call(
        functools.partial(_zzt_body, sigmoid=sigmoid),
        grid=(ma // bm,),
        in_specs=[
            pl.BlockSpec((bm, f), lambda i: (i, 0)),
            pl.BlockSpec((mb, f), lambda i: (0, 0)),
        ],
        out_specs=pl.BlockSpec((bm, mb), lambda i: (i, 0)),
        out_shape=jax.ShapeDtypeStruct((ma, mb), jnp.float32),
        interpret=_INTERPRET,
    )(za, zb)


_FE_KEYS = ('W1', 'b1', 'bn1_g', 'bn1_b', 'W2', 'b2', 'bn2_g', 'bn2_b')
_FF_KEYS = ('WQr', 'WKr', 'WVr', 'Wfcr', 'ln1_g', 'ln1_b', 'l1_W', 'l1_b',
            'ln2_g', 'ln2_b')
_TAIL_KEYS = ('ff_an_g', 'ff_an_b', 'ff_l1_W', 'ff_l1_b', 'ff_bn1_g',
              'ff_bn1_b', 'ff_l2_W', 'ff_l2_b', 'hd_W1', 'hd_b1', 'hd_bn_g',
              'hd_bn_b', 'hd_W2', 'hd_b2')


def _fe2_block(X, w):
    X = _bn(_gelu(jnp.dot(X, w['W1'], preferred_element_type=jnp.float32)
                  + w['b1']), w['bn1_g'], w['bn1_b'])
    X = _bn(_gelu(jnp.dot(X, w['W2'], preferred_element_type=jnp.float32)
                  + w['b2']), w['bn2_g'], w['bn2_b'])
    return X


def _mha_block(X, wq, wk, wv, wfc):
    out = None
    scale = np.float32(1.0 / np.sqrt(DK))
    for h in range(NH):
        Qh = jnp.dot(X, wq[h], preferred_element_type=jnp.float32)
        Kh = jnp.dot(X, wk[h], preferred_element_type=jnp.float32)
        Vh = jnp.dot(X, wv[h], preferred_element_type=jnp.float32)
        s = lax.dot_general(Qh, Kh, (((1,), (1,)), ((), ())),
                            preferred_element_type=jnp.float32) * scale
        s = s - jnp.max(s, axis=-1, keepdims=True)
        e = jnp.exp(s)
        attn = e / jnp.sum(e, axis=-1, keepdims=True)
        ctx = jnp.dot(attn, Vh, preferred_element_type=jnp.float32)
        o = jnp.dot(ctx, wfc[h], preferred_element_type=jnp.float32)
        out = o if out is None else out + o
    return out


def _decoder_body(*refs, names):
    r = {k: v for k, v in zip(names, refs[:len(names)])}
    out_x, out_cfv = refs[len(names):]
    dr2 = r['DR2'][...]
    ds2 = r['DS2'][...]
    X1 = jnp.concatenate([r['DR1'][...], ds2], axis=1)
    X2 = jnp.concatenate([dr2, r['DS1'][...]], axis=1)
    fe1 = {k: r['fe1_' + k][...] for k in _FE_KEYS}
    fe2 = {k: r['fe2_' + k][...] for k in _FE_KEYS}
    FD1 = _fe2_block(X1, fe1)
    FD2 = _fe2_block(X2, fe2)
    X = jnp.concatenate([dr2, FD1, FD2, ds2], axis=1)
    for l in range(NL):
        w = {k: r[f'ff{l}_' + k][...] for k in _FF_KEYS}
        a = _mha_block(X, w['WQr'], w['WKr'], w['WVr'], w['Wfcr'])
        X = _ln(a + X, w['ln1_g'], w['ln1_b'])
        o = jnp.dot(X, w['l1_W'], preferred_element_type=jnp.float32) + w['l1_b']
        X = _ln(o + X, w['ln2_g'], w['ln2_b'])
    t = {k: r[k][...] for k in _TAIL_KEYS}
    X1f = _ln(X, t['ff_an_g'], t['ff_an_b'])
    X2f = _bn(_gelu(jnp.dot(X1f, t['ff_l1_W'],
                            preferred_element_type=jnp.float32)
                    + t['ff_l1_b']), t['ff_bn1_g'], t['ff_bn1_b'])
    CFV = jnp.dot(X2f, t['ff_l2_W'],
                  preferred_element_type=jnp.float32) + t['ff_l2_b']
    Xh = jnp.concatenate([dr2, FD1, FD2, ds2, CFV], axis=1)
    Xh = _bn(_gelu(jnp.dot(Xh, t['hd_W1'],
                           preferred_element_type=jnp.float32)
                   + t['hd_b1']), t['hd_bn_g'], t['hd_bn_b'])
    out_x[...] = jnp.dot(Xh, t['hd_W2'],
                         preferred_element_type=jnp.float32) + t['hd_b2']
    out_cfv[...] = CFV


def _run_decoder(inputs):
    names = list(inputs.keys())
    vals = [inputs[k] for k in names]
    return pl.pallas_call(
        functools.partial(_decoder_body, names=names),
        in_specs=[pl.BlockSpec(v.shape, functools.partial(
            lambda nd, *_: (0,) * nd, v.ndim)) for v in vals],
        out_specs=[pl.BlockSpec((BATCH, 1), lambda *_: (0, 0)),
                   pl.BlockSpec((BATCH, DFUI // 4), lambda *_: (0, 0))],
        out_shape=[jax.ShapeDtypeStruct((BATCH, 1), jnp.float32),
                   jax.ShapeDtypeStruct((BATCH, DFUI // 4), jnp.float32)],
        interpret=_INTERPRET,
    )(*vals)


def _build_adj_deg(edge_index, edge_weight, n):
    src, dst = edge_index[0], edge_index[1]
    w = edge_weight if edge_weight is not None else jnp.ones(
        src.shape, jnp.float32)
    a = jnp.zeros((n, n), jnp.float32).at[dst, src].add(w)
    deg = jnp.zeros((n,), jnp.float32).at[dst].add(1.0)
    return a, deg


def _graph_encoder(p, pfx, a, deg, n):
    deg2d = deg.reshape(n, 1)
    y1 = _matmul(p[pfx + '_feat'], p[pfx + '_W1'])
    h1 = _sage_combine(a, y1, deg2d, p[pfx + '_b1'].reshape(1, -1), relu=True)
    y2 = _matmul(h1, p[pfx + '_W2'])
    z = _sage_combine(a, y2, deg2d, p[pfx + '_b2'].reshape(1, -1), relu=False)
    return z


def kernel(edge_index_dr, edge_weight_dr, edge_index_ds, edge_weight_ds,
           edge_index_drds, edges_ori, params):
    p = params
    a_dr, deg_dr = _build_adj_deg(edge_index_dr, edge_weight_dr, N_DR)
    a_ds, deg_ds = _build_adj_deg(edge_index_ds, edge_weight_ds, N_DS)
    a_dd, deg_dd = _build_adj_deg(edge_index_drds, None, N_DD)

    DR1a = _graph_encoder(p, 'dr', a_dr, deg_dr, N_DR)
    DS1a = _graph_encoder(p, 'ds', a_ds, deg_ds, N_DS)
    z = _graph_encoder(p, 'dd', a_dd, deg_dd, N_DD)
    z_dr = z[:N_DR]
    z_ds = z[N_DR:]

    rec_dr = _zzt(DR1a, DR1a)
    rec_ds = _zzt(DS1a, DS1a)
    rec_dd = _zzt(z_dr, z_ds, sigmoid=True)

    drug_id = edges_ori[:, 0]
    dis_id = edges_ori[:, 1]
    DR1 = DR1a[drug_id]
    DS1 = DS1a[dis_id]
    DR2 = z_dr[drug_id]
    DS2 = z_ds[dis_id]

    dec_in = {'DR1': DR1, 'DS1': DS1, 'DR2': DR2, 'DS2': DS2}
    for pfx in ('fe1', 'fe2'):
        for k in _FE_KEYS:
            v = p[pfx + '_' + k]
            dec_in[pfx + '_' + k] = v.reshape(1, -1) if v.ndim == 1 else v
    for l in range(NL):
        dec_in[f'ff{l}_WQr'] = p[f'ff{l}_WQ'].reshape(DFUI, NH, DK).transpose(1, 0, 2)
        dec_in[f'ff{l}_WKr'] = p[f'ff{l}_WK'].reshape(DFUI, NH, DK).transpose(1, 0, 2)
        dec_in[f'ff{l}_WVr'] = p[f'ff{l}_WV'].reshape(DFUI, NH, DK).transpose(1, 0, 2)
        dec_in[f'ff{l}_Wfcr'] = p[f'ff{l}_Wfc'].reshape(NH, DK, DFUI)
        for k in _FF_KEYS[4:]:
            v = p[f'ff{l}_' + k]
            dec_in[f'ff{l}_' + k] = v.reshape(1, -1) if v.ndim == 1 else v
    for k in _TAIL_KEYS:
        v = p[k]
        dec_in[k] = v.reshape(1, -1) if v.ndim == 1 else v

    out_x, cfv = _run_decoder(dec_in)
    return out_x, cfv, rec_dr, rec_ds, rec_dd
